# Initial kernel scaffold; baseline (speedup 1.0000x reference)
import jax
import jax.numpy as jnp
from jax.experimental import pallas as pl


def kernel(indices, weight):
    raise NotImplementedError("write your pallas kernel here")


# speedup vs baseline: 1.8459x; 1.8459x over previous
import functools

import jax
import jax.numpy as jnp
from jax import lax
from jax.experimental import pallas as pl
from jax.experimental.pallas import tpu as pltpu
from jax.experimental.pallas import tpu_sc as plsc

NC = 2
NS = 16
NW = NC * NS

IDXROW = 128
ROWS_PER_CHUNK = 1024
IDXROWS_PER_CHUNK = ROWS_PER_CHUNK // IDXROW


@functools.partial(jax.jit, static_argnames=("b", "d"))
def _emb_lookup(idx2d, weight, b, d):
    b_per_w = b // NW
    n_chunks = b_per_w // ROWS_PER_CHUNK
    mesh = plsc.VectorSubcoreMesh(core_axis_name="c", subcore_axis_name="s")

    @functools.partial(
        pl.kernel,
        mesh=mesh,
        out_type=jax.ShapeDtypeStruct((b, d), jnp.float32),
        compiler_params=pltpu.CompilerParams(use_tc_tiling_on_sc=False),
        scratch_types=[
            pltpu.VMEM((IDXROWS_PER_CHUNK, IDXROW), jnp.int32),
            pltpu.VMEM((ROWS_PER_CHUNK, d), jnp.float32),
            pltpu.SemaphoreType.DMA,
        ],
    )
    def k(idx_hbm, table_hbm, out_hbm, idx_v, rows_v, sem):
        wid = lax.axis_index("s") * NC + lax.axis_index("c")
        base_row = wid * b_per_w
        base_irow = base_row // IDXROW

        def body(i, carry):
            row_off = pl.multiple_of(base_row + i * ROWS_PER_CHUNK, 512)
            irow_off = pl.multiple_of(
                base_irow + i * IDXROWS_PER_CHUNK, 8)
            pltpu.sync_copy(
                idx_hbm.at[pl.ds(irow_off, IDXROWS_PER_CHUNK)], idx_v)
            copies = [
                pltpu.async_copy(
                    table_hbm.at[idx_v.at[j]],
                    rows_v.at[pl.ds(j * IDXROW, IDXROW)],
                    sem,
                )
                for j in range(IDXROWS_PER_CHUNK)
            ]
            for c in copies:
                c.wait()
            pltpu.sync_copy(
                rows_v, out_hbm.at[pl.ds(row_off, ROWS_PER_CHUNK)])
            return carry

        lax.fori_loop(0, n_chunks, body, 0)

    return k(idx2d, weight)


def kernel(indices, weight):
    batch, hist = indices.shape
    _, d = weight.shape
    b = batch * hist
    idx2d = indices.astype(jnp.int32).reshape(b // IDXROW, IDXROW)
    out = _emb_lookup(idx2d, weight, b, d)
    return out.reshape(batch, hist, d)


# speedup vs baseline: 1.8680x; 1.0120x over previous
import functools

import jax
import jax.numpy as jnp
from jax import lax
from jax.experimental import pallas as pl
from jax.experimental.pallas import tpu as pltpu
from jax.experimental.pallas import tpu_sc as plsc

NC = 2
NS = 16
NW = NC * NS

SAMPLES_PER_CHUNK = 16


@functools.partial(jax.jit, static_argnames=("batch", "hist", "d"))
def _emb_lookup(indices, weight, batch, hist, d):
    s_per_w = batch // NW
    n_chunks = s_per_w // SAMPLES_PER_CHUNK
    n_outer = n_chunks // 2
    mesh = plsc.VectorSubcoreMesh(core_axis_name="c", subcore_axis_name="s")

    @functools.partial(
        pl.kernel,
        mesh=mesh,
        out_type=jax.ShapeDtypeStruct((batch, hist, d), jnp.float32),
        compiler_params=pltpu.CompilerParams(use_tc_tiling_on_sc=False),
        scratch_types=(
            [pltpu.VMEM((SAMPLES_PER_CHUNK, hist), jnp.int32)
             for _ in range(2)]
            + [pltpu.VMEM((SAMPLES_PER_CHUNK, hist, d), jnp.float32)
               for _ in range(2)]
            + [pltpu.SemaphoreType.DMA for _ in range(4)]
        ),
    )
    def k(idx_hbm, table_hbm, out_hbm, *scratch):
        idx_v = scratch[0:2]
        rows_v = scratch[2:4]
        g_sem = scratch[4:6]
        o_sem = scratch[6:8]

        wid = lax.axis_index("s") * NC + lax.axis_index("c")
        base_s = wid * s_per_w

        def load_idx(c, s):
            off = pl.multiple_of(
                base_s + c * SAMPLES_PER_CHUNK, SAMPLES_PER_CHUNK)
            pltpu.sync_copy(idx_hbm.at[pl.ds(off, SAMPLES_PER_CHUNK)],
                            idx_v[s])

        def fire_gathers(s):
            for j in range(SAMPLES_PER_CHUNK):
                pltpu.async_copy(
                    table_hbm.at[idx_v[s].at[j]],
                    rows_v[s].at[j],
                    g_sem[s],
                )

        def drain_gathers(s):
            for j in range(SAMPLES_PER_CHUNK):
                pltpu.make_async_copy(
                    table_hbm.at[idx_v[s].at[j]],
                    rows_v[s].at[j],
                    g_sem[s],
                ).wait()

        def fire_store(c, s):
            off = pl.multiple_of(
                base_s + c * SAMPLES_PER_CHUNK, SAMPLES_PER_CHUNK)
            pltpu.async_copy(
                rows_v[s], out_hbm.at[pl.ds(off, SAMPLES_PER_CHUNK)],
                o_sem[s])

        def drain_store(c, s):
            off = pl.multiple_of(
                base_s + c * SAMPLES_PER_CHUNK, SAMPLES_PER_CHUNK)
            pltpu.make_async_copy(
                rows_v[s], out_hbm.at[pl.ds(off, SAMPLES_PER_CHUNK)],
                o_sem[s]).wait()

        for s in range(2):
            load_idx(s, s)
            fire_gathers(s)

        def body(g, carry):
            c0 = g * 2
            for s in range(2):
                drain_gathers(s)
                fire_store(c0 + s, s)

            for s in range(2):
                @pl.when(c0 + s + 2 < n_chunks)
                def _():
                    load_idx(c0 + s + 2, s)
                    drain_store(c0 + s, s)
                    fire_gathers(s)

                @pl.when(c0 + s + 2 >= n_chunks)
                def _():
                    drain_store(c0 + s, s)

            return carry

        lax.fori_loop(0, n_outer, body, 0)

    return k(indices, weight)


def kernel(indices, weight):
    batch, hist = indices.shape
    _, d = weight.shape
    return _emb_lookup(indices.astype(jnp.int32), weight, batch, hist, d)
